# baseline (speedup 1.0000x reference)
import functools

import jax
import jax.numpy as jnp
from jax import lax
from jax.experimental import pallas as pl
from jax.experimental.pallas import tpu as pltpu
from jax.experimental.pallas import tpu_sc as plsc

N = 10000
D = 128
E = 160000
H = 64

NC = 2
NS = 16
C = 128
GB = 8
NCHUNK = 80
EPT = NCHUNK * C
PAD_E = EPT * NS
ACC_N = 10112
ROWS_PT = ACC_N // NS


@functools.cache
def _sc_agg():
  def body(table, src_all, dst_all, zfeat, out_sums, acc, src_v, dst_v, rows_v):
    c = lax.axis_index("c")
    s = lax.axis_index("s")

    r0 = pl.multiple_of(s * ROWS_PT, 8)
    pltpu.sync_copy(zfeat.at[pl.ds(r0, ROWS_PT)], acc.at[pl.ds(r0, ROWS_PT)])
    plsc.subcore_barrier()

    base = s * EPT

    def chunk(ci, carry):
      off = pl.multiple_of(base + ci * C, C)
      pltpu.sync_copy(src_all.at[c, pl.ds(off, C)], src_v)
      pltpu.sync_copy(dst_all.at[c, pl.ds(off, C)], dst_v)
      pltpu.sync_copy(table.at[src_v], rows_v)
      pltpu.sync_copy(rows_v, acc.at[dst_v], add=True)
      return carry

    lax.fori_loop(0, NCHUNK, chunk, 0)
    plsc.subcore_barrier()

    pltpu.sync_copy(acc.at[pl.ds(r0, ROWS_PT)], out_sums.at[c, pl.ds(r0, ROWS_PT)])

  mesh = plsc.VectorSubcoreMesh(core_axis_name="c", subcore_axis_name="s")
  return pl.kernel(
      body,
      out_type=[jax.ShapeDtypeStruct((NC, ACC_N, D), jnp.float32)],
      mesh=mesh,
      scratch_types=[
          pltpu.VMEM_SHARED((ACC_N, D), jnp.float32),
          pltpu.VMEM((C,), jnp.int32),
          pltpu.VMEM((C,), jnp.int32),
          pltpu.VMEM((C, D), jnp.float32),
      ])


@functools.cache
def _sc_deg():
  def body(idx5, zfeat, ones_c, out_deg, acc, idxblk, ones_v, ssem):
    c = lax.axis_index("c")
    s = lax.axis_index("s")

    r0 = pl.multiple_of(s * ROWS_PT, 8)
    pltpu.sync_copy(zfeat.at[pl.ds(r0, ROWS_PT)], acc.at[pl.ds(r0, ROWS_PT)])
    pltpu.sync_copy(ones_c, ones_v)
    plsc.subcore_barrier()

    def group(gi, carry):
      pltpu.sync_copy(idx5.at[c, s, pl.ds(gi * GB, GB)], idxblk)
      sd = [pltpu.async_copy(ones_v, acc.at[idxblk.at[k, 1]], ssem, add=True)
            for k in range(GB)]
      for d in sd:
        d.wait()
      return carry

    lax.fori_loop(0, NCHUNK // GB, group, 0)
    plsc.subcore_barrier()

    pltpu.sync_copy(acc.at[pl.ds(r0, ROWS_PT)], out_deg.at[c, pl.ds(r0, ROWS_PT)])

  mesh = plsc.VectorSubcoreMesh(core_axis_name="c", subcore_axis_name="s")
  return pl.kernel(
      body,
      out_type=[jax.ShapeDtypeStruct((NC, ACC_N, D), jnp.float32)],
      mesh=mesh,
      scratch_types=[
          pltpu.VMEM_SHARED((ACC_N, D), jnp.float32),
          pltpu.VMEM((GB, 2, C), jnp.int32),
          pltpu.VMEM((C, D), jnp.float32),
          pltpu.SemaphoreType.DMA,
      ])


_BN = 1000


def _tc_base_body(sp, sn, dp, dn, x, wp, bp, wn, bn, out):
  aggp = sp[...] / jnp.maximum(dp[:, 0:1], 1.0)
  aggn = sn[...] / jnp.maximum(dn[:, 0:1], 1.0)
  xb = x[...]
  out[:, 0:H] = jnp.tanh(aggp @ wp[0:D] + xb @ wp[D:2 * D] + bp[...])
  out[:, H:2 * H] = jnp.tanh(aggn @ wn[0:D] + xb @ wn[D:2 * D] + bn[...])


def _tc_deep_body(sp, sn, dp, dn, hcat, wp, bp, wn, bn, out):
  aggp = sp[...] / jnp.maximum(dp[:, 0:1], 1.0)
  aggn = sn[...] / jnp.maximum(dn[:, 0:1], 1.0)
  hb = hcat[...]
  out[:, 0:H] = jnp.tanh(
      aggp[:, 0:H] @ wp[0:H] + aggn[:, H:2 * H] @ wp[H:2 * H]
      + hb[:, 0:H] @ wp[2 * H:3 * H] + bp[...])
  out[:, H:2 * H] = jnp.tanh(
      aggn[:, 0:H] @ wn[H:2 * H] + aggp[:, H:2 * H] @ wn[0:H]
      + hb[:, H:2 * H] @ wn[2 * H:3 * H] + bn[...])


def _row_block(feat):
  return pl.BlockSpec((_BN, feat), lambda i: (i, 0))


def _full_block(shape):
  return pl.BlockSpec(shape, lambda i: (0,) * len(shape))


def _make_tc(body, kdim):
  return pl.pallas_call(
      body,
      grid=(N // _BN,),
      in_specs=[
          _row_block(D), _row_block(D), _row_block(D), _row_block(D),
          _row_block(D),
          _full_block((kdim, H)), _full_block((1, H)),
          _full_block((kdim, H)), _full_block((1, H)),
      ],
      out_specs=_row_block(D),
      out_shape=jax.ShapeDtypeStruct((N, D), jnp.float32),
  )


_tc_base = _make_tc(_tc_base_body, 2 * D)
_tc_deep = _make_tc(_tc_deep_body, 3 * H)


def kernel(positive_edges, negative_edges, pos_adj, neg_adj, X,
           W_pos_base, b_pos_base, W_neg_base, b_neg_base,
           W_pos_deep, b_pos_deep, W_neg_deep, b_neg_deep):
  del pos_adj, neg_adj
  pad = PAD_E - E
  pad_src = jnp.zeros((1, pad), jnp.int32)
  pad_dst = jnp.full((1, pad), N, jnp.int32)
  src_all = jnp.concatenate([jnp.stack([positive_edges[0], negative_edges[0]]),
                             jnp.broadcast_to(pad_src, (NC, pad))], axis=1)
  dst_all = jnp.concatenate([jnp.stack([positive_edges[1], negative_edges[1]]),
                             jnp.broadcast_to(pad_dst, (NC, pad))], axis=1)
  idx5 = jnp.stack([src_all.reshape(NC, NS, NCHUNK, C),
                    dst_all.reshape(NC, NS, NCHUNK, C)], axis=3)

  zfeat = jnp.zeros((ACC_N, D), jnp.float32)
  ones_c = jnp.ones((C, D), jnp.float32)

  (deg,) = _sc_deg()(idx5, zfeat, ones_c)
  (sums1,) = _sc_agg()(X, src_all, dst_all, zfeat)
  hcat = _tc_base(sums1[0], sums1[1], deg[0], deg[1], X,
                  W_pos_base, b_pos_base.reshape(1, H),
                  W_neg_base, b_neg_base.reshape(1, H))
  (sums2,) = _sc_agg()(hcat, src_all, dst_all, zfeat)
  z = _tc_deep(sums2[0], sums2[1], deg[0], deg[1], hcat,
               W_pos_deep, b_pos_deep.reshape(1, H),
               W_neg_deep, b_neg_deep.reshape(1, H))
  return z


# speedup vs baseline: 1.2615x; 1.2615x over previous
import functools

import jax
import jax.numpy as jnp
from jax import lax
from jax.experimental import pallas as pl
from jax.experimental.pallas import tpu as pltpu
from jax.experimental.pallas import tpu_sc as plsc

N = 10000
D = 128
E = 160000
H = 64

NC = 2
NS = 16
C = 128
EPT = 10112
NCHUNK = EPT // C
PAD_E = EPT * NS
ACC_N = 10112
ROWS_PT = ACC_N // NS


@functools.cache
def _sc_agg():
  def body(table, src_all, dst_all, zfeat, out_sums, acc, src_v, dst_v, rows_v):
    c = lax.axis_index("c")
    s = lax.axis_index("s")

    r0 = pl.multiple_of(s * ROWS_PT, 8)
    pltpu.sync_copy(zfeat.at[pl.ds(r0, ROWS_PT)], acc.at[pl.ds(r0, ROWS_PT)])
    plsc.subcore_barrier()

    base = s * EPT

    def chunk(ci, carry):
      off = pl.multiple_of(base + ci * C, C)
      pltpu.sync_copy(src_all.at[c, pl.ds(off, C)], src_v)
      pltpu.sync_copy(dst_all.at[c, pl.ds(off, C)], dst_v)
      pltpu.sync_copy(table.at[src_v], rows_v)
      pltpu.sync_copy(rows_v, acc.at[dst_v], add=True)
      return carry

    lax.fori_loop(0, NCHUNK, chunk, 0)
    plsc.subcore_barrier()

    pltpu.sync_copy(acc.at[pl.ds(r0, ROWS_PT)], out_sums.at[c, pl.ds(r0, ROWS_PT)])

  mesh = plsc.VectorSubcoreMesh(core_axis_name="c", subcore_axis_name="s")
  return pl.kernel(
      body,
      out_type=[jax.ShapeDtypeStruct((NC, ACC_N, D), jnp.float32)],
      mesh=mesh,
      scratch_types=[
          pltpu.VMEM_SHARED((ACC_N, D), jnp.float32),
          pltpu.VMEM((C,), jnp.int32),
          pltpu.VMEM((C,), jnp.int32),
          pltpu.VMEM((C, D), jnp.float32),
      ])


@functools.cache
def _sc_deg():
  def body(dst_all, zfeat, ones_c, out_deg, acc, dst_v, ones_v):
    c = lax.axis_index("c")
    s = lax.axis_index("s")

    r0 = pl.multiple_of(s * ROWS_PT, 8)
    pltpu.sync_copy(zfeat.at[pl.ds(r0, ROWS_PT)], acc.at[pl.ds(r0, ROWS_PT)])
    pltpu.sync_copy(ones_c, ones_v)
    plsc.subcore_barrier()

    base = s * EPT

    def chunk(ci, carry):
      off = pl.multiple_of(base + ci * C, C)
      pltpu.sync_copy(dst_all.at[c, pl.ds(off, C)], dst_v)
      pltpu.sync_copy(ones_v, acc.at[dst_v], add=True)
      return carry

    lax.fori_loop(0, NCHUNK, chunk, 0)
    plsc.subcore_barrier()

    pltpu.sync_copy(acc.at[pl.ds(r0, ROWS_PT)], out_deg.at[c, pl.ds(r0, ROWS_PT)])

  mesh = plsc.VectorSubcoreMesh(core_axis_name="c", subcore_axis_name="s")
  return pl.kernel(
      body,
      out_type=[jax.ShapeDtypeStruct((NC, ACC_N, D), jnp.float32)],
      mesh=mesh,
      scratch_types=[
          pltpu.VMEM_SHARED((ACC_N, D), jnp.float32),
          pltpu.VMEM((C,), jnp.int32),
          pltpu.VMEM((C, D), jnp.float32),
      ])


_BN = 1000


def _tc_base_body(sp, sn, dp, dn, x, wp, bp, wn, bn, out):
  aggp = sp[...] / jnp.maximum(dp[:, 0:1], 1.0)
  aggn = sn[...] / jnp.maximum(dn[:, 0:1], 1.0)
  xb = x[...]
  out[:, 0:H] = jnp.tanh(aggp @ wp[0:D] + xb @ wp[D:2 * D] + bp[...])
  out[:, H:2 * H] = jnp.tanh(aggn @ wn[0:D] + xb @ wn[D:2 * D] + bn[...])


def _tc_deep_body(sp, sn, dp, dn, hcat, wp, bp, wn, bn, out):
  aggp = sp[...] / jnp.maximum(dp[:, 0:1], 1.0)
  aggn = sn[...] / jnp.maximum(dn[:, 0:1], 1.0)
  hb = hcat[...]
  out[:, 0:H] = jnp.tanh(
      aggp[:, 0:H] @ wp[0:H] + aggn[:, H:2 * H] @ wp[H:2 * H]
      + hb[:, 0:H] @ wp[2 * H:3 * H] + bp[...])
  out[:, H:2 * H] = jnp.tanh(
      aggn[:, 0:H] @ wn[H:2 * H] + aggp[:, H:2 * H] @ wn[0:H]
      + hb[:, H:2 * H] @ wn[2 * H:3 * H] + bn[...])


def _row_block(feat):
  return pl.BlockSpec((_BN, feat), lambda i: (i, 0))


def _full_block(shape):
  return pl.BlockSpec(shape, lambda i: (0,) * len(shape))


def _make_tc(body, kdim):
  return pl.pallas_call(
      body,
      grid=(N // _BN,),
      in_specs=[
          _row_block(D), _row_block(D), _row_block(D), _row_block(D),
          _row_block(D),
          _full_block((kdim, H)), _full_block((1, H)),
          _full_block((kdim, H)), _full_block((1, H)),
      ],
      out_specs=_row_block(D),
      out_shape=jax.ShapeDtypeStruct((N, D), jnp.float32),
  )


_tc_base = _make_tc(_tc_base_body, 2 * D)
_tc_deep = _make_tc(_tc_deep_body, 3 * H)


def kernel(positive_edges, negative_edges, pos_adj, neg_adj, X,
           W_pos_base, b_pos_base, W_neg_base, b_neg_base,
           W_pos_deep, b_pos_deep, W_neg_deep, b_neg_deep):
  del pos_adj, neg_adj
  pad = PAD_E - E
  pad_src = jnp.zeros((1, pad), jnp.int32)
  pad_dst = jnp.full((1, pad), N, jnp.int32)
  srcs = jnp.stack([positive_edges[0], negative_edges[0]])
  dsts = jnp.stack([positive_edges[1], negative_edges[1]])
  src_all = jnp.concatenate([srcs, jnp.broadcast_to(pad_src, (NC, pad))], axis=1)
  dst_all = jnp.concatenate([dsts, jnp.broadcast_to(pad_dst, (NC, pad))], axis=1)

  zfeat = jnp.zeros((ACC_N, D), jnp.float32)
  ones_c = jnp.ones((C, D), jnp.float32)

  (deg,) = _sc_deg()(dst_all, zfeat, ones_c)
  (sums1,) = _sc_agg()(X, src_all, dst_all, zfeat)
  hcat = _tc_base(sums1[0], sums1[1], deg[0], deg[1], X,
                  W_pos_base, b_pos_base.reshape(1, H),
                  W_neg_base, b_neg_base.reshape(1, H))
  (sums2,) = _sc_agg()(hcat, src_all, dst_all, zfeat)
  z = _tc_deep(sums2[0], sums2[1], deg[0], deg[1], hcat,
               W_pos_deep, b_pos_deep.reshape(1, H),
               W_neg_deep, b_neg_deep.reshape(1, H))
  return z
